# baseline (speedup 1.0000x reference)
import jax
import jax.numpy as jnp
from jax import lax
from jax.experimental import pallas as pl
from jax.experimental.pallas import tpu as pltpu
from jax.experimental.pallas import tpu_sc as plsc

B = 512
NV = 4096
L = 16
NC = 2
NS = 16
NW = NC * NS
BPW = B // NW
LIM = 0.125


def _sc_body(off_hbm, ph_hbm, th_hbm, ts_hbm, out_hbm,
             off_buf, ph_buf, th_buf, ts_buf, res_buf):
    wid = lax.axis_index("s") * NC + lax.axis_index("c")
    base = wid * BPW
    pltpu.sync_copy(ts_hbm.at[pl.ds(base, BPW)], ts_buf)

    wbase = lax.iota(jnp.int32, L).astype(jnp.float32) * 0.125 - 1.0

    tot = jnp.zeros((L,), jnp.float32)
    cnt = jnp.zeros((L,), jnp.float32)
    for i in range(BPW):
        b = base + i
        pltpu.sync_copy(off_hbm.at[b], off_buf)
        pltpu.sync_copy(ph_hbm.at[b], ph_buf)
        pltpu.sync_copy(th_hbm.at[b], th_buf)
        tsv = ts_buf[i, :]
        ts0 = tsv[0]
        ts1 = tsv[1]
        ts2 = tsv[2]
        t2v = jnp.clip(ts2 - wbase, -LIM, LIM)

        def chunk(j, carry, ts0=ts0, ts1=ts1, t2v=t2v):
            tot, cnt = carry
            d = (j >> 4).astype(jnp.float32)
            h = (j & 15).astype(jnp.float32)
            v0 = j * L
            t0 = jnp.clip(ts0 - (d * 0.125 - 1.0), -LIM, LIM)
            t1 = jnp.clip(ts1 - (h * 0.125 - 1.0), -LIM, LIM)
            o0 = off_buf[0, pl.ds(v0, L)]
            o1 = off_buf[1, pl.ds(v0, L)]
            o2 = off_buf[2, pl.ds(v0, L)]
            p0 = ph_buf[0, pl.ds(v0, L)]
            p1 = ph_buf[1, pl.ds(v0, L)]
            tt = th_buf[pl.ds(v0, L)]
            m = jnp.logical_and(p1 > p0, tt >= 0.5)
            s = jnp.abs(o0 - t0) + jnp.abs(o1 - t1) + jnp.abs(o2 - t2v)
            return (tot + jnp.where(m, s, 0.0),
                    cnt + jnp.where(m, 1.0, 0.0))

        tot, cnt = lax.fori_loop(0, NV // L, chunk, (tot, cnt))

    res_buf[0, :] = tot
    res_buf[1, :] = cnt
    pltpu.sync_copy(res_buf, out_hbm.at[wid])


def kernel(offsets, target_skeleton, predicted_heatmap, target_heatmap):
    off = offsets.reshape(B, 3, NV)
    ph = predicted_heatmap.reshape(B, 2, NV)
    th = target_heatmap.reshape(B, NV)
    ts = jnp.pad(target_skeleton.reshape(B, 3), ((0, 0), (0, L - 3)))

    mesh = plsc.VectorSubcoreMesh(core_axis_name="c", subcore_axis_name="s")
    f = pl.kernel(
        _sc_body,
        out_type=jax.ShapeDtypeStruct((NW, 2, L), jnp.float32),
        mesh=mesh,
        scratch_types=[
            pltpu.VMEM((3, NV), jnp.float32),
            pltpu.VMEM((2, NV), jnp.float32),
            pltpu.VMEM((NV,), jnp.float32),
            pltpu.VMEM((BPW, L), jnp.float32),
            pltpu.VMEM((2, L), jnp.float32),
        ],
    )
    out = f(off, ph, th, ts)
    tot = jnp.sum(out[:, 0, :])
    cnt = jnp.sum(out[:, 1, :])
    denom = jnp.maximum(cnt * 3.0, 1.0)
    return jnp.where(cnt > 0, tot / denom, jnp.float32(0.0))


# speedup vs baseline: 1.3573x; 1.3573x over previous
import jax
import jax.numpy as jnp
from jax import lax
from jax.experimental import pallas as pl
from jax.experimental.pallas import tpu as pltpu
from jax.experimental.pallas import tpu_sc as plsc

B = 512
NV = 4096
L = 16
NC = 2
NS = 16
NW = NC * NS
BPW = B // NW
LIM = 0.125


def _start(off_hbm, ph_hbm, th_hbm, b, off_buf, ph_buf, th_buf, sem):
    return (
        pltpu.async_copy(off_hbm.at[b], off_buf, sem),
        pltpu.async_copy(ph_hbm.at[b], ph_buf, sem),
        pltpu.async_copy(th_hbm.at[b], th_buf, sem),
    )


def _accumulate(off_buf, ph_buf, th_buf, tsv, wbase, accs):
    ts0, ts1, ts2 = tsv[0], tsv[1], tsv[2]
    t2v = jnp.clip(ts2 - wbase, -LIM, LIM)

    def group(i, accs, ts0=ts0, ts1=ts1, t2v=t2v):
        a = list(accs)
        d = (i >> 4).astype(jnp.float32)
        t0 = jnp.clip(ts0 - (d * 0.125 - 1.0), -LIM, LIM)
        hb = (i & 15).astype(jnp.float32)
        for k in range(4):
            t1 = jnp.clip(ts1 - ((hb + float(k)) * 0.125 - 1.0), -LIM, LIM)
            v0 = (i + k) * L
            o0 = off_buf[0, pl.ds(v0, L)]
            o1 = off_buf[1, pl.ds(v0, L)]
            o2 = off_buf[2, pl.ds(v0, L)]
            p0 = ph_buf[0, pl.ds(v0, L)]
            p1 = ph_buf[1, pl.ds(v0, L)]
            tt = th_buf[pl.ds(v0, L)]
            m = jnp.logical_and(p1 > p0, tt >= 0.5)
            s = jnp.abs(o0 - t0) + jnp.abs(o1 - t1) + jnp.abs(o2 - t2v)
            a[k] = a[k] + jnp.where(m, s, 0.0)
            a[4 + k] = a[4 + k] + jnp.where(m, 1.0, 0.0)
        return tuple(a)

    return plsc.parallel_loop(0, NV // L, 4, carry=accs)(group)


def _sc_body(off_hbm, ph_hbm, th_hbm, ts_hbm, out_hbm,
             off0, ph0, th0, off1, ph1, th1, ts_buf, res_buf, sem0, sem1):
    wid = lax.axis_index("s") * NC + lax.axis_index("c")
    base = wid * BPW
    pltpu.sync_copy(ts_hbm.at[pl.ds(base, BPW)], ts_buf)

    wbase = lax.iota(jnp.int32, L).astype(jnp.float32) * 0.125 - 1.0

    zero = jnp.zeros((L,), jnp.float32)
    accs = (zero,) * 8

    slots = ((off0, ph0, th0, sem0), (off1, ph1, th1, sem1))
    pending = [None, None]
    pending[0] = _start(off_hbm, ph_hbm, th_hbm, base, *slots[0])
    for i in range(BPW):
        s = i % 2
        if i + 1 < BPW:
            pending[1 - s] = _start(off_hbm, ph_hbm, th_hbm, base + i + 1,
                                    *slots[1 - s])
        for cp in pending[s]:
            cp.wait()
        obuf, pbuf, tbuf, _ = slots[s]
        accs = _accumulate(obuf, pbuf, tbuf, ts_buf[i, :], wbase, accs)

    tot = (accs[0] + accs[1]) + (accs[2] + accs[3])
    cnt = (accs[4] + accs[5]) + (accs[6] + accs[7])
    res_buf[0, :] = tot
    res_buf[1, :] = cnt
    pltpu.sync_copy(res_buf, out_hbm.at[wid])


def kernel(offsets, target_skeleton, predicted_heatmap, target_heatmap):
    off = offsets.reshape(B, 3, NV)
    ph = predicted_heatmap.reshape(B, 2, NV)
    th = target_heatmap.reshape(B, NV)
    ts = jnp.pad(target_skeleton.reshape(B, 3), ((0, 0), (0, L - 3)))

    mesh = plsc.VectorSubcoreMesh(core_axis_name="c", subcore_axis_name="s")
    f = pl.kernel(
        _sc_body,
        out_type=jax.ShapeDtypeStruct((NW, 2, L), jnp.float32),
        mesh=mesh,
        scratch_types=[
            pltpu.VMEM((3, NV), jnp.float32),
            pltpu.VMEM((2, NV), jnp.float32),
            pltpu.VMEM((NV,), jnp.float32),
            pltpu.VMEM((3, NV), jnp.float32),
            pltpu.VMEM((2, NV), jnp.float32),
            pltpu.VMEM((NV,), jnp.float32),
            pltpu.VMEM((BPW, L), jnp.float32),
            pltpu.VMEM((2, L), jnp.float32),
            pltpu.SemaphoreType.DMA,
            pltpu.SemaphoreType.DMA,
        ],
    )
    out = f(off, ph, th, ts)
    tot = jnp.sum(out[:, 0, :])
    cnt = jnp.sum(out[:, 1, :])
    denom = jnp.maximum(cnt * 3.0, 1.0)
    return jnp.where(cnt > 0, tot / denom, jnp.float32(0.0))
